# baseline (speedup 1.0000x reference)
import functools

import jax
import jax.numpy as jnp
from jax.experimental import pallas as pl
from jax.experimental.pallas import tpu as pltpu


def _decoder_kernel(B, N, HID, s_ref, h0i_ref, w1g_ref, b1g_ref,
                    w1c_ref, b1c_ref, w2g_ref, b2g_ref, w2c_ref, b2c_ref,
                    wpt_ref, bp_ref, out_ref, h0_scr, h1_scr, cur_scr,
                    y_scr, z_scr):
    t = pl.program_id(0)

    @pl.when(t == 0)
    def _init():
        h0_scr[...] = h0i_ref[0]
        h1_scr[...] = h0i_ref[1]
        cur_scr[...] = jnp.zeros((8, B * N), jnp.float32)

    def matmul(a, b):
        return jax.lax.dot(a, b, preferred_element_type=jnp.float32)

    def apply_s(scr, lo, F):
        return jnp.concatenate(
            [matmul(scr[lo:lo + F, b * N:(b + 1) * N], s_ref[...])
             for b in range(B)], axis=1)

    def cell(inp, h, F, wg_ref, bg_ref, wc_ref, bc_ref, scr_y, scr_z):
        y0 = jnp.concatenate([h, inp], axis=0)
        scr_y[0:F] = y0.astype(jnp.bfloat16)
        y1 = apply_s(scr_y, 0, F)
        scr_y[F:2 * F] = y1.astype(jnp.bfloat16)
        y2 = 2.0 * apply_s(scr_y, F, F) - y0
        scr_y[2 * F:3 * F] = y2.astype(jnp.bfloat16)
        g = jax.nn.sigmoid(matmul(wg_ref[...], scr_y[0:3 * F])
                           + bg_ref[...])
        r, u = g[:HID], g[HID:]
        rh = r * h
        scr_z[0:HID] = rh.astype(jnp.bfloat16)
        scr_z[HID:F] = scr_y[HID:F]
        z1 = apply_s(scr_z, 0, HID)
        scr_z[F:F + HID] = z1.astype(jnp.bfloat16)
        scr_z[F + HID:2 * F] = scr_y[F + HID:2 * F]
        z2 = 2.0 * apply_s(scr_z, F, HID) - rh
        scr_z[2 * F:2 * F + HID] = z2.astype(jnp.bfloat16)
        scr_z[2 * F + HID:3 * F] = scr_y[2 * F + HID:3 * F]
        c = jnp.tanh(matmul(wc_ref[...], scr_z[0:3 * F]) + bc_ref[...])
        return u * h + (1.0 - u) * c

    h0 = cell(cur_scr[...], h0_scr[...], HID + 8, w1g_ref, b1g_ref,
              w1c_ref, b1c_ref, y_scr, z_scr)
    h0_scr[...] = h0
    h1 = cell(h0, h1_scr[...], 2 * HID, w2g_ref, b2g_ref,
              w2c_ref, b2c_ref, y_scr, z_scr)
    h1_scr[...] = h1

    proj = matmul(wpt_ref[...], h1) + bp_ref[...]
    cur_scr[0:1] = proj
    out_ref[0] = proj


def kernel(inputs, initial_hidden_state, supports, W1_gate, b1_gate,
           W1_cand, b1_cand, W2_gate, b2_gate, W2_cand, b2_cand, Wp, bp):
    seq_len, B = inputs.shape[0], inputs.shape[1]
    N = supports.shape[1]
    HID = Wp.shape[0]
    OUT_DIM = Wp.shape[1]
    num_layers = initial_hidden_state.shape[0]
    nm = 3

    S = supports[0].T.astype(jnp.bfloat16)
    h0i = (initial_hidden_state.reshape(num_layers, B, N, HID)
           .transpose(0, 3, 1, 2).reshape(num_layers, HID, B * N))

    def prep_w(W, in_rows, out_cols, h_lo, h_hi, pad_rows):
        w = W.reshape(in_rows, nm, out_cols)
        hpart = w[h_lo:h_hi]
        ipart = jnp.concatenate(
            [w[:h_lo], w[h_hi:],
             jnp.zeros((pad_rows, nm, out_cols), jnp.float32)], axis=0)
        blk = jnp.concatenate([hpart, ipart], axis=0)
        wcat = blk.transpose(1, 0, 2).reshape(-1, out_cols).T
        return wcat.astype(jnp.bfloat16)

    w1gcat = prep_w(W1_gate, 1 + HID, 2 * HID, 1, 1 + HID, 7)
    w1ccat = prep_w(W1_cand, 1 + HID, HID, 1, 1 + HID, 7)
    w2gcat = prep_w(W2_gate, 2 * HID, 2 * HID, HID, 2 * HID, 0)
    w2ccat = prep_w(W2_cand, 2 * HID, HID, HID, 2 * HID, 0)

    b1g = b1_gate.reshape(2 * HID, 1)
    b1c = b1_cand.reshape(HID, 1)
    b2g = b2_gate.reshape(2 * HID, 1)
    b2c = b2_cand.reshape(HID, 1)
    wpt = Wp.T
    bp2 = bp.reshape(1, 1)

    body = functools.partial(_decoder_kernel, B, N, HID)
    full = lambda shape: pl.BlockSpec(shape, lambda t: (0,) * len(shape))
    out = pl.pallas_call(
        body,
        grid=(seq_len,),
        in_specs=[
            full(S.shape), full(h0i.shape),
            full(w1gcat.shape), full(b1g.shape),
            full(w1ccat.shape), full(b1c.shape),
            full(w2gcat.shape), full(b2g.shape),
            full(w2ccat.shape), full(b2c.shape),
            full(wpt.shape), full(bp2.shape),
        ],
        out_specs=pl.BlockSpec((1, 1, B * N), lambda t: (t, 0, 0)),
        out_shape=jax.ShapeDtypeStruct((seq_len, 1, B * N), jnp.float32),
        scratch_shapes=[
            pltpu.VMEM((HID, B * N), jnp.float32),
            pltpu.VMEM((HID, B * N), jnp.float32),
            pltpu.VMEM((8, B * N), jnp.float32),
            pltpu.VMEM((3 * 2 * HID, B * N), jnp.bfloat16),
            pltpu.VMEM((3 * 2 * HID, B * N), jnp.bfloat16),
        ],
        compiler_params=pltpu.CompilerParams(
            dimension_semantics=("arbitrary",),
        ),
    )(S, h0i, w1gcat, b1g, w1ccat, b1c, w2gcat, b2g, w2ccat, b2c, wpt, bp2)

    return out.reshape(seq_len, B, N * OUT_DIM)


# speedup vs baseline: 1.0074x; 1.0074x over previous
import functools

import jax
import jax.numpy as jnp
from jax.experimental import pallas as pl
from jax.experimental.pallas import tpu as pltpu


def _decoder_kernel(B, N, HID, s_ref, h0i_ref, w1g_ref, b1g_ref,
                    w1c_ref, b1c_ref, w2g_ref, b2g_ref, w2c_ref, b2c_ref,
                    wpt_ref, bp_ref, out_ref, h0_scr, h1_scr, cur_scr,
                    y_scr, z_scr, s_scr):
    t = pl.program_id(0)

    @pl.when(t == 0)
    def _init():
        s_scr[...] = s_ref[...].T.astype(jnp.bfloat16)
        for b in range(B):
            h0_scr[:, b * N:(b + 1) * N] = h0i_ref[0, b].T
            h1_scr[:, b * N:(b + 1) * N] = h0i_ref[1, b].T
        cur_scr[...] = jnp.zeros((8, B * N), jnp.float32)

    def matmul(a, b):
        return jax.lax.dot(a, b, preferred_element_type=jnp.float32)

    def apply_s(scr, lo, F):
        return jnp.concatenate(
            [matmul(scr[lo:lo + F, b * N:(b + 1) * N], s_scr[...])
             for b in range(B)], axis=1)

    def cell(inp, h, F, wg_ref, bg_ref, wc_ref, bc_ref, scr_y, scr_z):
        y0 = jnp.concatenate([h, inp], axis=0)
        scr_y[0:F] = y0.astype(jnp.bfloat16)
        y1 = apply_s(scr_y, 0, F)
        scr_y[F:2 * F] = y1.astype(jnp.bfloat16)
        y2 = 2.0 * apply_s(scr_y, F, F) - y0
        scr_y[2 * F:3 * F] = y2.astype(jnp.bfloat16)
        g = jax.nn.sigmoid(matmul(wg_ref[...], scr_y[0:3 * F])
                           + bg_ref[...])
        r, u = g[:HID], g[HID:]
        z0 = jnp.concatenate([r * h, inp], axis=0)
        scr_z[0:F] = z0.astype(jnp.bfloat16)
        z1 = apply_s(scr_z, 0, F)
        scr_z[F:2 * F] = z1.astype(jnp.bfloat16)
        z2 = 2.0 * apply_s(scr_z, F, F) - z0
        scr_z[2 * F:3 * F] = z2.astype(jnp.bfloat16)
        c = jnp.tanh(matmul(wc_ref[...], scr_z[0:3 * F]) + bc_ref[...])
        return u * h + (1.0 - u) * c

    h0 = cell(cur_scr[...], h0_scr[...], HID + 8, w1g_ref, b1g_ref,
              w1c_ref, b1c_ref, y_scr, z_scr)
    h0_scr[...] = h0
    h1 = cell(h0, h1_scr[...], 2 * HID, w2g_ref, b2g_ref,
              w2c_ref, b2c_ref, y_scr, z_scr)
    h1_scr[...] = h1

    proj = matmul(wpt_ref[...], h1) + bp_ref[...]
    cur_scr[0:1] = proj
    out_ref[0] = proj


def kernel(inputs, initial_hidden_state, supports, W1_gate, b1_gate,
           W1_cand, b1_cand, W2_gate, b2_gate, W2_cand, b2_cand, Wp, bp):
    seq_len, B = inputs.shape[0], inputs.shape[1]
    N = supports.shape[1]
    HID = Wp.shape[0]
    OUT_DIM = Wp.shape[1]
    num_layers = initial_hidden_state.shape[0]
    nm = 3

    S = supports[0]
    h0i = initial_hidden_state.reshape(num_layers, B, N, HID)

    def prep_w(W, in_rows, out_cols, h_lo, h_hi, pad_rows):
        w = W.reshape(in_rows, nm, out_cols)
        hpart = w[h_lo:h_hi]
        ipart = jnp.concatenate(
            [w[:h_lo], w[h_hi:],
             jnp.zeros((pad_rows, nm, out_cols), jnp.float32)], axis=0)
        blk = jnp.concatenate([hpart, ipart], axis=0)
        wcat = blk.transpose(1, 0, 2).reshape(-1, out_cols).T
        return wcat.astype(jnp.bfloat16)

    w1gcat = prep_w(W1_gate, 1 + HID, 2 * HID, 1, 1 + HID, 7)
    w1ccat = prep_w(W1_cand, 1 + HID, HID, 1, 1 + HID, 7)
    w2gcat = prep_w(W2_gate, 2 * HID, 2 * HID, HID, 2 * HID, 0)
    w2ccat = prep_w(W2_cand, 2 * HID, HID, HID, 2 * HID, 0)

    b1g = b1_gate.reshape(2 * HID, 1)
    b1c = b1_cand.reshape(HID, 1)
    b2g = b2_gate.reshape(2 * HID, 1)
    b2c = b2_cand.reshape(HID, 1)
    wpt = Wp.T
    bp2 = bp.reshape(1, 1)

    body = functools.partial(_decoder_kernel, B, N, HID)
    full = lambda shape: pl.BlockSpec(shape, lambda t: (0,) * len(shape))
    out = pl.pallas_call(
        body,
        grid=(seq_len,),
        in_specs=[
            full(S.shape), full(h0i.shape),
            full(w1gcat.shape), full(b1g.shape),
            full(w1ccat.shape), full(b1c.shape),
            full(w2gcat.shape), full(b2g.shape),
            full(w2ccat.shape), full(b2c.shape),
            full(wpt.shape), full(bp2.shape),
        ],
        out_specs=pl.BlockSpec((1, 1, B * N), lambda t: (t, 0, 0)),
        out_shape=jax.ShapeDtypeStruct((seq_len, 1, B * N), jnp.float32),
        scratch_shapes=[
            pltpu.VMEM((HID, B * N), jnp.float32),
            pltpu.VMEM((HID, B * N), jnp.float32),
            pltpu.VMEM((8, B * N), jnp.float32),
            pltpu.VMEM((3 * 2 * HID, B * N), jnp.bfloat16),
            pltpu.VMEM((3 * 2 * HID, B * N), jnp.bfloat16),
            pltpu.VMEM((N, N), jnp.bfloat16),
        ],
        compiler_params=pltpu.CompilerParams(
            dimension_semantics=("arbitrary",),
        ),
    )(S, h0i, w1gcat, b1g, w1ccat, b1c, w2gcat, b2g, w2ccat, b2c, wpt, bp2)

    return out.reshape(seq_len, B, N * OUT_DIM)


# speedup vs baseline: 1.0381x; 1.0305x over previous
import functools

import jax
import jax.numpy as jnp
from jax.experimental import pallas as pl
from jax.experimental.pallas import tpu as pltpu


def _decoder_kernel(B, N, HID, s_ref, h0i_ref, w1g_ref, b1g_ref,
                    w1c_ref, b1c_ref, w2g_ref, b2g_ref, w2c_ref, b2c_ref,
                    wpt_ref, bp_ref, out_ref, h0_scr, h1_scr, cur_scr,
                    y_scr, z_scr):
    t = pl.program_id(0)

    @pl.when(t == 0)
    def _init():
        h0_scr[...] = h0i_ref[0]
        h1_scr[...] = h0i_ref[1]
        cur_scr[...] = jnp.zeros((8, B * N), jnp.float32)

    def matmul(a, b):
        return jax.lax.dot(a, b, preferred_element_type=jnp.float32)

    def apply_s(scr, lo, F):
        return jnp.concatenate(
            [matmul(scr[lo:lo + F, b * N:(b + 1) * N], s_ref[...])
             for b in range(B)], axis=1)

    def cell(inp, h, F, wg_ref, bg_ref, wc_ref, bc_ref, scr_y, scr_z):
        y0 = jnp.concatenate([h, inp], axis=0)
        scr_y[0:F] = y0.astype(jnp.bfloat16)
        y1 = apply_s(scr_y, 0, F)
        scr_y[F:2 * F] = y1.astype(jnp.bfloat16)
        y2 = 2.0 * apply_s(scr_y, F, F) - y0
        scr_y[2 * F:3 * F] = y2.astype(jnp.bfloat16)
        g = jax.nn.sigmoid(matmul(wg_ref[...], scr_y[0:3 * F])
                           + bg_ref[...])
        r, u = g[:HID], g[HID:]
        z0 = jnp.concatenate([r * h, inp], axis=0)
        scr_z[0:F] = z0.astype(jnp.bfloat16)
        z1 = apply_s(scr_z, 0, F)
        scr_z[F:2 * F] = z1.astype(jnp.bfloat16)
        z2 = 2.0 * apply_s(scr_z, F, F) - z0
        scr_z[2 * F:3 * F] = z2.astype(jnp.bfloat16)
        c = jnp.tanh(matmul(wc_ref[...], scr_z[0:3 * F]) + bc_ref[...])
        return u * h + (1.0 - u) * c

    for step in range(2):
        h0 = cell(cur_scr[...], h0_scr[...], HID + 8, w1g_ref, b1g_ref,
                  w1c_ref, b1c_ref, y_scr, z_scr)
        h0_scr[...] = h0
        h1 = cell(h0, h1_scr[...], 2 * HID, w2g_ref, b2g_ref,
                  w2c_ref, b2c_ref, y_scr, z_scr)
        h1_scr[...] = h1

        proj = matmul(wpt_ref[...], h1) + bp_ref[...]
        cur_scr[0:1] = proj
        out_ref[step] = proj


def kernel(inputs, initial_hidden_state, supports, W1_gate, b1_gate,
           W1_cand, b1_cand, W2_gate, b2_gate, W2_cand, b2_cand, Wp, bp):
    seq_len, B = inputs.shape[0], inputs.shape[1]
    N = supports.shape[1]
    HID = Wp.shape[0]
    OUT_DIM = Wp.shape[1]
    num_layers = initial_hidden_state.shape[0]
    nm = 3

    S = supports[0].T.astype(jnp.bfloat16)
    h0i = (initial_hidden_state.reshape(num_layers, B, N, HID)
           .transpose(0, 3, 1, 2).reshape(num_layers, HID, B * N))

    def prep_w(W, in_rows, out_cols, h_lo, h_hi, pad_rows):
        w = W.reshape(in_rows, nm, out_cols)
        hpart = w[h_lo:h_hi]
        ipart = jnp.concatenate(
            [w[:h_lo], w[h_hi:],
             jnp.zeros((pad_rows, nm, out_cols), jnp.float32)], axis=0)
        blk = jnp.concatenate([hpart, ipart], axis=0)
        wcat = blk.transpose(1, 0, 2).reshape(-1, out_cols).T
        return wcat.astype(jnp.bfloat16)

    w1gcat = prep_w(W1_gate, 1 + HID, 2 * HID, 1, 1 + HID, 7)
    w1ccat = prep_w(W1_cand, 1 + HID, HID, 1, 1 + HID, 7)
    w2gcat = prep_w(W2_gate, 2 * HID, 2 * HID, HID, 2 * HID, 0)
    w2ccat = prep_w(W2_cand, 2 * HID, HID, HID, 2 * HID, 0)

    b1g = b1_gate.reshape(2 * HID, 1)
    b1c = b1_cand.reshape(HID, 1)
    b2g = b2_gate.reshape(2 * HID, 1)
    b2c = b2_cand.reshape(HID, 1)
    wpt = Wp.T
    bp2 = bp.reshape(1, 1)

    body = functools.partial(_decoder_kernel, B, N, HID)
    full = lambda shape: pl.BlockSpec(shape, lambda t: (0,) * len(shape))
    out = pl.pallas_call(
        body,
        grid=(seq_len // 2,),
        in_specs=[
            full(S.shape), full(h0i.shape),
            full(w1gcat.shape), full(b1g.shape),
            full(w1ccat.shape), full(b1c.shape),
            full(w2gcat.shape), full(b2g.shape),
            full(w2ccat.shape), full(b2c.shape),
            full(wpt.shape), full(bp2.shape),
        ],
        out_specs=pl.BlockSpec((2, 1, B * N), lambda t: (t, 0, 0)),
        out_shape=jax.ShapeDtypeStruct((seq_len, 1, B * N), jnp.float32),
        scratch_shapes=[
            pltpu.VMEM((HID, B * N), jnp.float32),
            pltpu.VMEM((HID, B * N), jnp.float32),
            pltpu.VMEM((8, B * N), jnp.float32),
            pltpu.VMEM((3 * 2 * HID, B * N), jnp.bfloat16),
            pltpu.VMEM((3 * 2 * HID, B * N), jnp.bfloat16),
        ],
        compiler_params=pltpu.CompilerParams(
            dimension_semantics=("arbitrary",),
        ),
    )(S, h0i, w1gcat, b1g, w1ccat, b1c, w2gcat, b2g, w2ccat, b2c, wpt, bp2)

    return out.reshape(seq_len, B, N * OUT_DIM)
